# baseline (device time: 92440 ns/iter reference)
import jax
import jax.numpy as jnp
from jax import lax
from jax.experimental import pallas as pl
from jax.experimental.pallas import tpu as pltpu

N_DEV = 4
SUB = 2


def kernel(x, w_mat, scale_x, scale_w):
    m_total, k_loc = x.shape
    k_loc2, n = w_mat.shape
    assert k_loc == k_loc2
    m_per = m_total // N_DEV
    half = n // 2
    rows = m_per // SUB

    CW_SLOT = (3, 2, 1)
    CCW_SLOT = (1, 2, 3)

    def body(x_ref, w_ref, sx_ref, sw_ref, out_ref,
             stage, wq_ref, send_cw, send_ccw, recv_cw, recv_ccw,
             stage_sems, send_sems_cw, send_sems_ccw,
             recv_sems_cw, recv_sems_ccw):
        my = lax.axis_index("i")
        left = lax.rem(my + N_DEV - 1, N_DEV)
        right = lax.rem(my + 1, N_DEV)

        copies = {}
        for j in (3, 1, 2, 0):
            c = lax.rem(my + j, N_DEV)
            cp = pltpu.make_async_copy(
                x_ref.at[pl.ds(c * m_per, m_per)],
                stage.at[j],
                stage_sems.at[j],
            )
            cp.start()
            copies[j] = cp

        def partial(slot, lo):
            xb = stage[slot, :, :].astype(jnp.float8_e4m3fn)
            return jnp.dot(xb, wq_ref[:, lo:lo + half],
                           preferred_element_type=jnp.float32)

        def stage_sends(q, p_cw, p_ccw):
            for j in range(SUB):
                lo = j * rows
                send_cw[q, j, :, :] = p_cw[lo:lo + rows, :].astype(jnp.bfloat16)
                send_ccw[q, j, :, :] = p_ccw[lo:lo + rows, :].astype(jnp.bfloat16)

        wq_ref[:, :] = w_ref[:, :].astype(jnp.float8_e4m3fn)
        copies[3].wait()
        copies[1].wait()
        stage_sends(0, partial(CW_SLOT[0], 0), partial(CCW_SLOT[0], half))

        barrier_sem = pltpu.get_barrier_semaphore()
        for nbr in (left, right):
            pl.semaphore_signal(
                barrier_sem, inc=1,
                device_id=(nbr,), device_id_type=pl.DeviceIdType.MESH,
            )
        pl.semaphore_wait(barrier_sem, 2)

        prev_cw = prev_ccw = None
        for s in range(N_DEV - 1):
            q = s % 2
            descs_cw, descs_ccw = [], []
            for j in range(SUB):
                if s > 0:
                    prev_cw[j].wait()
                    send_cw[q, j, :, :] = (
                        send_cw[q, j, :, :] + recv_cw[s - 1, j, :, :])
                r = pltpu.make_async_remote_copy(
                    src_ref=send_cw.at[q, j], dst_ref=recv_cw.at[s, j],
                    send_sem=send_sems_cw.at[q, j],
                    recv_sem=recv_sems_cw.at[s, j],
                    device_id=(right,), device_id_type=pl.DeviceIdType.MESH,
                )
                r.start()
                descs_cw.append(r)
                if s > 0:
                    prev_ccw[j].wait()
                    send_ccw[q, j, :, :] = (
                        send_ccw[q, j, :, :] + recv_ccw[s - 1, j, :, :])
                r = pltpu.make_async_remote_copy(
                    src_ref=send_ccw.at[q, j], dst_ref=recv_ccw.at[s, j],
                    send_sem=send_sems_ccw.at[q, j],
                    recv_sem=recv_sems_ccw.at[s, j],
                    device_id=(left,), device_id_type=pl.DeviceIdType.MESH,
                )
                r.start()
                descs_ccw.append(r)
            prev_cw, prev_ccw = descs_cw, descs_ccw

            if s < N_DEV - 2:
                if CW_SLOT[s + 1] == 2:
                    copies[2].wait()
                stage_sends(1 - q, partial(CW_SLOT[s + 1], 0),
                            partial(CCW_SLOT[s + 1], half))
            else:
                copies[0].wait()
                p_cw = partial(0, 0)
                p_ccw = partial(0, half)

        scale = sx_ref[0] * sw_ref[0]
        for j in range(SUB):
            lo = j * rows
            prev_cw[j].wait()
            out_ref[lo:lo + rows, :half] = (
                p_cw[lo:lo + rows, :]
                + recv_cw[N_DEV - 2, j, :, :].astype(jnp.float32)) * scale
            prev_ccw[j].wait()
            out_ref[lo:lo + rows, half:] = (
                p_ccw[lo:lo + rows, :]
                + recv_ccw[N_DEV - 2, j, :, :].astype(jnp.float32)) * scale

    send = pltpu.VMEM((2, SUB, rows, half), jnp.bfloat16)
    recv = pltpu.VMEM((N_DEV - 1, SUB, rows, half), jnp.bfloat16)
    send_sems = pltpu.SemaphoreType.DMA((2, SUB))
    recv_sems = pltpu.SemaphoreType.DMA((N_DEV - 1, SUB))
    return pl.pallas_call(
        body,
        out_shape=jax.ShapeDtypeStruct((m_per, n), jnp.float32),
        in_specs=[
            pl.BlockSpec(memory_space=pl.ANY),
            pl.BlockSpec(memory_space=pltpu.VMEM),
            pl.BlockSpec(memory_space=pltpu.SMEM),
            pl.BlockSpec(memory_space=pltpu.SMEM),
        ],
        out_specs=pl.BlockSpec(memory_space=pltpu.VMEM),
        scratch_shapes=[
            pltpu.VMEM((N_DEV, m_per, k_loc), jnp.float32),
            pltpu.VMEM((k_loc, n), jnp.float8_e4m3fn),
            send, send, recv, recv,
            pltpu.SemaphoreType.DMA((N_DEV,)),
            send_sems, send_sems, recv_sems, recv_sems,
        ],
        compiler_params=pltpu.CompilerParams(
            collective_id=0,
            vmem_limit_bytes=62 * 1024 * 1024,
        ),
    )(x, w_mat, scale_x, scale_w)


# device time: 89804 ns/iter; 1.0294x vs baseline; 1.0294x over previous
import jax
import jax.numpy as jnp
from jax import lax
from jax.experimental import pallas as pl
from jax.experimental.pallas import tpu as pltpu

N_DEV = 4
SUB = 2


def kernel(x, w_mat, scale_x, scale_w):
    m_total, k_loc = x.shape
    k_loc2, n = w_mat.shape
    assert k_loc == k_loc2
    m_per = m_total // N_DEV
    half = n // 2
    rows = m_per // SUB

    CW_SLOT = (3, 2, 1)
    CCW_SLOT = (1, 2, 3)

    def body(x_ref, w_ref, sx_ref, sw_ref, out_ref,
             stage, wq_ref, send_cw, send_ccw, recv_cw, recv_ccw,
             stage_sems, send_sems_cw, send_sems_ccw,
             recv_sems_cw, recv_sems_ccw):
        my = lax.axis_index("i")
        left = lax.rem(my + N_DEV - 1, N_DEV)
        right = lax.rem(my + 1, N_DEV)

        barrier_sem = pltpu.get_barrier_semaphore()
        for nbr in (left, right):
            pl.semaphore_signal(
                barrier_sem, inc=1,
                device_id=(nbr,), device_id_type=pl.DeviceIdType.MESH,
            )

        copies = {}
        for j in (3, 1, 2, 0):
            c = lax.rem(my + j, N_DEV)
            cp = pltpu.make_async_copy(
                x_ref.at[pl.ds(c * m_per, m_per)],
                stage.at[j],
                stage_sems.at[j],
            )
            cp.start()
            copies[j] = cp

        def partial(slot, lo):
            xb = stage[slot, :, :].astype(jnp.float8_e4m3fn)
            return jnp.dot(xb, wq_ref[:, lo:lo + half],
                           preferred_element_type=jnp.float32)

        def stage_sends(q, p_cw, p_ccw):
            for j in range(SUB):
                lo = j * rows
                send_cw[q, j, :, :] = p_cw[lo:lo + rows, :].astype(jnp.bfloat16)
                send_ccw[q, j, :, :] = p_ccw[lo:lo + rows, :].astype(jnp.bfloat16)

        def partial_rows(slot, lo, r0):
            xb = stage[slot, r0:r0 + rows, :].astype(jnp.float8_e4m3fn)
            return jnp.dot(xb, wq_ref[:, lo:lo + half],
                           preferred_element_type=jnp.float32)

        def start_rdma(dir_send, dir_recv, dir_ssems, dir_rsems, q, s, j, dev):
            r = pltpu.make_async_remote_copy(
                src_ref=dir_send.at[q, j], dst_ref=dir_recv.at[s, j],
                send_sem=dir_ssems.at[q, j], recv_sem=dir_rsems.at[s, j],
                device_id=(dev,), device_id_type=pl.DeviceIdType.MESH,
            )
            r.start()
            return r

        wq_ref[:, :] = w_ref[:, :].astype(jnp.float8_e4m3fn)
        copies[3].wait()
        copies[1].wait()

        pl.semaphore_wait(barrier_sem, 2)
        prev_cw, prev_ccw = [None, None], [None, None]
        for j in range(SUB):
            send_cw[0, j, :, :] = (
                partial_rows(CW_SLOT[0], 0, j * rows).astype(jnp.bfloat16))
            prev_cw[j] = start_rdma(
                send_cw, recv_cw, send_sems_cw, recv_sems_cw, 0, 0, j, right)
            send_ccw[0, j, :, :] = (
                partial_rows(CCW_SLOT[0], half, j * rows).astype(jnp.bfloat16))
            prev_ccw[j] = start_rdma(
                send_ccw, recv_ccw, send_sems_ccw, recv_sems_ccw, 0, 0, j, left)

        for s in range(1, N_DEV - 1):
            q = s % 2
            if s == 1:
                copies[2].wait()
                stage_sends(1, partial(CW_SLOT[1], 0), partial(CCW_SLOT[1], half))
            descs_cw, descs_ccw = [], []
            for j in range(SUB):
                prev_cw[j].wait()
                send_cw[q, j, :, :] = (
                    send_cw[q, j, :, :] + recv_cw[s - 1, j, :, :])
                descs_cw.append(start_rdma(
                    send_cw, recv_cw, send_sems_cw, recv_sems_cw,
                    q, s, j, right))
                prev_ccw[j].wait()
                send_ccw[q, j, :, :] = (
                    send_ccw[q, j, :, :] + recv_ccw[s - 1, j, :, :])
                descs_ccw.append(start_rdma(
                    send_ccw, recv_ccw, send_sems_ccw, recv_sems_ccw,
                    q, s, j, left))
            prev_cw, prev_ccw = descs_cw, descs_ccw

            if s < N_DEV - 2:
                if CW_SLOT[s + 1] == 2:
                    copies[2].wait()
                stage_sends(1 - q, partial(CW_SLOT[s + 1], 0),
                            partial(CCW_SLOT[s + 1], half))
            else:
                copies[0].wait()
                p_cw = partial(0, 0)
                p_ccw = partial(0, half)

        scale = sx_ref[0] * sw_ref[0]
        for j in range(SUB):
            lo = j * rows
            prev_cw[j].wait()
            out_ref[lo:lo + rows, :half] = (
                p_cw[lo:lo + rows, :]
                + recv_cw[N_DEV - 2, j, :, :].astype(jnp.float32)) * scale
            prev_ccw[j].wait()
            out_ref[lo:lo + rows, half:] = (
                p_ccw[lo:lo + rows, :]
                + recv_ccw[N_DEV - 2, j, :, :].astype(jnp.float32)) * scale

    send = pltpu.VMEM((2, SUB, rows, half), jnp.bfloat16)
    recv = pltpu.VMEM((N_DEV - 1, SUB, rows, half), jnp.bfloat16)
    send_sems = pltpu.SemaphoreType.DMA((2, SUB))
    recv_sems = pltpu.SemaphoreType.DMA((N_DEV - 1, SUB))
    return pl.pallas_call(
        body,
        out_shape=jax.ShapeDtypeStruct((m_per, n), jnp.float32),
        in_specs=[
            pl.BlockSpec(memory_space=pl.ANY),
            pl.BlockSpec(memory_space=pltpu.VMEM),
            pl.BlockSpec(memory_space=pltpu.SMEM),
            pl.BlockSpec(memory_space=pltpu.SMEM),
        ],
        out_specs=pl.BlockSpec(memory_space=pltpu.VMEM),
        scratch_shapes=[
            pltpu.VMEM((N_DEV, m_per, k_loc), jnp.float32),
            pltpu.VMEM((k_loc, n), jnp.float8_e4m3fn),
            send, send, recv, recv,
            pltpu.SemaphoreType.DMA((N_DEV,)),
            send_sems, send_sems, recv_sems, recv_sems,
        ],
        compiler_params=pltpu.CompilerParams(
            collective_id=0,
            vmem_limit_bytes=62 * 1024 * 1024,
        ),
    )(x, w_mat, scale_x, scale_w)


# device time: 88005 ns/iter; 1.0504x vs baseline; 1.0204x over previous
import jax
import jax.numpy as jnp
from jax import lax
from jax.experimental import pallas as pl
from jax.experimental.pallas import tpu as pltpu

N_DEV = 4
SUB = 2


def kernel(x, w_mat, scale_x, scale_w):
    m_total, k_loc = x.shape
    k_loc2, n = w_mat.shape
    assert k_loc == k_loc2
    m_per = m_total // N_DEV
    half = n // 2
    rows = m_per // SUB

    CW_SLOT = (3, 2, 1)
    CCW_SLOT = (1, 2, 3)

    def body(x_ref, w_ref, sx_ref, sw_ref, out_ref,
             stage, wq_ref, send_cw, send_ccw, recv_cw, recv_ccw,
             stage_sems, send_sems_cw, send_sems_ccw,
             recv_sems_cw, recv_sems_ccw):
        my = lax.axis_index("i")
        left = lax.rem(my + N_DEV - 1, N_DEV)
        right = lax.rem(my + 1, N_DEV)

        barrier_sem = pltpu.get_barrier_semaphore()
        for nbr in (left, right):
            pl.semaphore_signal(
                barrier_sem, inc=1,
                device_id=(nbr,), device_id_type=pl.DeviceIdType.MESH,
            )

        copies = {}
        for j in (3, 1, 2, 0):
            c = lax.rem(my + j, N_DEV)
            cp = pltpu.make_async_copy(
                x_ref.at[pl.ds(c * m_per, m_per)],
                stage.at[j],
                stage_sems.at[j],
            )
            cp.start()
            copies[j] = cp

        def partial(slot, lo):
            xb = stage[slot, :, :].astype(jnp.float8_e4m3fn)
            return jnp.dot(xb, wq_ref[:, lo:lo + half],
                           preferred_element_type=jnp.float32)

        def stage_sends(q, p_cw, p_ccw):
            for j in range(SUB):
                lo = j * rows
                send_cw[q, j, :, :] = p_cw[lo:lo + rows, :].astype(jnp.bfloat16)
                send_ccw[q, j, :, :] = p_ccw[lo:lo + rows, :].astype(jnp.bfloat16)

        def partial_rows(slot, lo, r0):
            xb = stage[slot, r0:r0 + rows, :].astype(jnp.float8_e4m3fn)
            return jnp.dot(xb, wq_ref[:, lo:lo + half],
                           preferred_element_type=jnp.float32)

        def start_rdma(dir_send, dir_recv, dir_ssems, dir_rsems, q, s, j, dev):
            r = pltpu.make_async_remote_copy(
                src_ref=dir_send.at[q, j], dst_ref=dir_recv.at[s, j],
                send_sem=dir_ssems.at[q, j], recv_sem=dir_rsems.at[s, j],
                device_id=(dev,), device_id_type=pl.DeviceIdType.MESH,
            )
            r.start()
            return r

        wq_ref[:, :] = w_ref[:, :].astype(jnp.float8_e4m3fn)
        copies[3].wait()
        copies[1].wait()

        pl.semaphore_wait(barrier_sem, 2)
        prev_cw, prev_ccw = [None, None], [None, None]
        for j in range(SUB):
            send_cw[0, j, :, :] = (
                partial_rows(CW_SLOT[0], 0, j * rows).astype(jnp.bfloat16))
            prev_cw[j] = start_rdma(
                send_cw, recv_cw, send_sems_cw, recv_sems_cw, 0, 0, j, right)
            send_ccw[0, j, :, :] = (
                partial_rows(CCW_SLOT[0], half, j * rows).astype(jnp.bfloat16))
            prev_ccw[j] = start_rdma(
                send_ccw, recv_ccw, send_sems_ccw, recv_sems_ccw, 0, 0, j, left)

        for s in range(1, N_DEV - 1):
            q = s % 2
            if s == 1:
                copies[2].wait()
                stage_sends(1, partial(CW_SLOT[1], 0), partial(CCW_SLOT[1], half))
            descs_cw, descs_ccw = [], []
            for j in range(SUB):
                prev_cw[j].wait()
                send_cw[q, j, :, :] = (
                    send_cw[q, j, :, :] + recv_cw[s - 1, j, :, :])
                descs_cw.append(start_rdma(
                    send_cw, recv_cw, send_sems_cw, recv_sems_cw,
                    q, s, j, right))
                prev_ccw[j].wait()
                send_ccw[q, j, :, :] = (
                    send_ccw[q, j, :, :] + recv_ccw[s - 1, j, :, :])
                descs_ccw.append(start_rdma(
                    send_ccw, recv_ccw, send_sems_ccw, recv_sems_ccw,
                    q, s, j, left))
            prev_cw, prev_ccw = descs_cw, descs_ccw

            if s < N_DEV - 2:
                if CW_SLOT[s + 1] == 2:
                    copies[2].wait()
                stage_sends(1 - q, partial(CW_SLOT[s + 1], 0),
                            partial(CCW_SLOT[s + 1], half))
            else:
                copies[0].wait()
                p_cw = partial(0, 0)
                p_ccw = partial(0, half)

        scale = sx_ref[0] * sw_ref[0]
        out_dmas = []
        for j in range(SUB):
            lo = j * rows
            prev_cw[j].wait()
            stage[3, lo:lo + rows, :] = (
                p_cw[lo:lo + rows, :]
                + recv_cw[N_DEV - 2, j, :, :].astype(jnp.float32)) * scale
            cp = pltpu.make_async_copy(
                stage.at[3, pl.ds(lo, rows)],
                out_ref.at[pl.ds(lo, rows), pl.ds(0, half)],
                stage_sems.at[j],
            )
            cp.start()
            out_dmas.append(cp)
            prev_ccw[j].wait()
            stage[1, lo:lo + rows, :] = (
                p_ccw[lo:lo + rows, :]
                + recv_ccw[N_DEV - 2, j, :, :].astype(jnp.float32)) * scale
            cp = pltpu.make_async_copy(
                stage.at[1, pl.ds(lo, rows)],
                out_ref.at[pl.ds(lo, rows), pl.ds(half, half)],
                stage_sems.at[SUB + j],
            )
            cp.start()
            out_dmas.append(cp)
        for cp in out_dmas:
            cp.wait()

    send = pltpu.VMEM((2, SUB, rows, half), jnp.bfloat16)
    recv = pltpu.VMEM((N_DEV - 1, SUB, rows, half), jnp.bfloat16)
    send_sems = pltpu.SemaphoreType.DMA((2, SUB))
    recv_sems = pltpu.SemaphoreType.DMA((N_DEV - 1, SUB))
    return pl.pallas_call(
        body,
        out_shape=jax.ShapeDtypeStruct((m_per, n), jnp.float32),
        in_specs=[
            pl.BlockSpec(memory_space=pl.ANY),
            pl.BlockSpec(memory_space=pltpu.VMEM),
            pl.BlockSpec(memory_space=pltpu.SMEM),
            pl.BlockSpec(memory_space=pltpu.SMEM),
        ],
        out_specs=pl.BlockSpec(memory_space=pl.ANY),
        scratch_shapes=[
            pltpu.VMEM((N_DEV, m_per, k_loc), jnp.float32),
            pltpu.VMEM((k_loc, n), jnp.float8_e4m3fn),
            send, send, recv, recv,
            pltpu.SemaphoreType.DMA((N_DEV,)),
            send_sems, send_sems, recv_sems, recv_sems,
        ],
        compiler_params=pltpu.CompilerParams(
            collective_id=0,
            vmem_limit_bytes=62 * 1024 * 1024,
        ),
    )(x, w_mat, scale_x, scale_w)


# device time: 87092 ns/iter; 1.0614x vs baseline; 1.0105x over previous
import jax
import jax.numpy as jnp
from jax import lax
from jax.experimental import pallas as pl
from jax.experimental.pallas import tpu as pltpu

N_DEV = 4
SUB = 4


def kernel(x, w_mat, scale_x, scale_w):
    m_total, k_loc = x.shape
    k_loc2, n = w_mat.shape
    assert k_loc == k_loc2
    m_per = m_total // N_DEV
    half = n // 2
    rows = m_per // SUB

    CW_SLOT = (3, 2, 1)
    CCW_SLOT = (1, 2, 3)

    def body(x_ref, w_ref, sx_ref, sw_ref, out_ref,
             stage, wq_ref, send_cw, send_ccw, recv_cw, recv_ccw,
             stage_sems, send_sems_cw, send_sems_ccw,
             recv_sems_cw, recv_sems_ccw):
        my = lax.axis_index("i")
        left = lax.rem(my + N_DEV - 1, N_DEV)
        right = lax.rem(my + 1, N_DEV)

        barrier_sem = pltpu.get_barrier_semaphore()
        for nbr in (left, right):
            pl.semaphore_signal(
                barrier_sem, inc=1,
                device_id=(nbr,), device_id_type=pl.DeviceIdType.MESH,
            )

        copies = {}
        for j in (3, 1, 2, 0):
            c = lax.rem(my + j, N_DEV)
            cp = pltpu.make_async_copy(
                x_ref.at[pl.ds(c * m_per, m_per)],
                stage.at[j],
                stage_sems.at[j],
            )
            cp.start()
            copies[j] = cp

        def partial(slot, lo):
            xb = stage[slot, :, :].astype(jnp.float8_e4m3fn)
            return jnp.dot(xb, wq_ref[:, lo:lo + half],
                           preferred_element_type=jnp.float32)

        def stage_sends(q, p_cw, p_ccw):
            for j in range(SUB):
                lo = j * rows
                send_cw[q, j, :, :] = p_cw[lo:lo + rows, :].astype(jnp.bfloat16)
                send_ccw[q, j, :, :] = p_ccw[lo:lo + rows, :].astype(jnp.bfloat16)

        def partial_rows(slot, lo, r0):
            xb = stage[slot, r0:r0 + rows, :].astype(jnp.float8_e4m3fn)
            return jnp.dot(xb, wq_ref[:, lo:lo + half],
                           preferred_element_type=jnp.float32)

        def start_rdma(dir_send, dir_recv, dir_ssems, dir_rsems, q, s, j, dev):
            r = pltpu.make_async_remote_copy(
                src_ref=dir_send.at[q, j], dst_ref=dir_recv.at[s, j],
                send_sem=dir_ssems.at[q, j], recv_sem=dir_rsems.at[s, j],
                device_id=(dev,), device_id_type=pl.DeviceIdType.MESH,
            )
            r.start()
            return r

        wq_ref[:, :] = w_ref[:, :].astype(jnp.float8_e4m3fn)
        copies[3].wait()
        copies[1].wait()

        pl.semaphore_wait(barrier_sem, 2)
        prev_cw, prev_ccw = [None] * SUB, [None] * SUB
        for j in range(SUB):
            send_cw[0, j, :, :] = (
                partial_rows(CW_SLOT[0], 0, j * rows).astype(jnp.bfloat16))
            prev_cw[j] = start_rdma(
                send_cw, recv_cw, send_sems_cw, recv_sems_cw, 0, 0, j, right)
            send_ccw[0, j, :, :] = (
                partial_rows(CCW_SLOT[0], half, j * rows).astype(jnp.bfloat16))
            prev_ccw[j] = start_rdma(
                send_ccw, recv_ccw, send_sems_ccw, recv_sems_ccw, 0, 0, j, left)

        for s in range(1, N_DEV - 1):
            q = s % 2
            if s == 1:
                copies[2].wait()
                stage_sends(1, partial(CW_SLOT[1], 0), partial(CCW_SLOT[1], half))
            descs_cw, descs_ccw = [], []
            for j in range(SUB):
                prev_cw[j].wait()
                send_cw[q, j, :, :] = (
                    send_cw[q, j, :, :] + recv_cw[s - 1, j, :, :])
                descs_cw.append(start_rdma(
                    send_cw, recv_cw, send_sems_cw, recv_sems_cw,
                    q, s, j, right))
                prev_ccw[j].wait()
                send_ccw[q, j, :, :] = (
                    send_ccw[q, j, :, :] + recv_ccw[s - 1, j, :, :])
                descs_ccw.append(start_rdma(
                    send_ccw, recv_ccw, send_sems_ccw, recv_sems_ccw,
                    q, s, j, left))
            prev_cw, prev_ccw = descs_cw, descs_ccw

            if s < N_DEV - 2:
                if CW_SLOT[s + 1] == 2:
                    copies[2].wait()
                stage_sends(1 - q, partial(CW_SLOT[s + 1], 0),
                            partial(CCW_SLOT[s + 1], half))
            else:
                copies[0].wait()
                p_cw = partial(0, 0)
                p_ccw = partial(0, half)

        scale = sx_ref[0] * sw_ref[0]
        out_dmas = []
        for j in range(SUB):
            lo = j * rows
            prev_cw[j].wait()
            stage[3, lo:lo + rows, :] = (
                p_cw[lo:lo + rows, :]
                + recv_cw[N_DEV - 2, j, :, :].astype(jnp.float32)) * scale
            cp = pltpu.make_async_copy(
                stage.at[3, pl.ds(lo, rows)],
                out_ref.at[pl.ds(lo, rows), pl.ds(0, half)],
                stage_sems.at[j],
            )
            cp.start()
            out_dmas.append(cp)
            prev_ccw[j].wait()
            stage[1, lo:lo + rows, :] = (
                p_ccw[lo:lo + rows, :]
                + recv_ccw[N_DEV - 2, j, :, :].astype(jnp.float32)) * scale
            cp = pltpu.make_async_copy(
                stage.at[1, pl.ds(lo, rows)],
                out_ref.at[pl.ds(lo, rows), pl.ds(half, half)],
                stage_sems.at[SUB + j],
            )
            cp.start()
            out_dmas.append(cp)
        for cp in out_dmas:
            cp.wait()

    send = pltpu.VMEM((2, SUB, rows, half), jnp.bfloat16)
    recv = pltpu.VMEM((N_DEV - 1, SUB, rows, half), jnp.bfloat16)
    send_sems = pltpu.SemaphoreType.DMA((2, SUB))
    recv_sems = pltpu.SemaphoreType.DMA((N_DEV - 1, SUB))
    return pl.pallas_call(
        body,
        out_shape=jax.ShapeDtypeStruct((m_per, n), jnp.float32),
        in_specs=[
            pl.BlockSpec(memory_space=pl.ANY),
            pl.BlockSpec(memory_space=pltpu.VMEM),
            pl.BlockSpec(memory_space=pltpu.SMEM),
            pl.BlockSpec(memory_space=pltpu.SMEM),
        ],
        out_specs=pl.BlockSpec(memory_space=pl.ANY),
        scratch_shapes=[
            pltpu.VMEM((N_DEV, m_per, k_loc), jnp.float32),
            pltpu.VMEM((k_loc, n), jnp.float8_e4m3fn),
            send, send, recv, recv,
            pltpu.SemaphoreType.DMA((N_DEV,)),
            send_sems, send_sems, recv_sems, recv_sems,
        ],
        compiler_params=pltpu.CompilerParams(
            collective_id=0,
            vmem_limit_bytes=62 * 1024 * 1024,
        ),
    )(x, w_mat, scale_x, scale_w)
